# baseline (speedup 1.0000x reference)
import jax
import jax.numpy as jnp
from jax.experimental import pallas as pl
from jax.experimental.pallas import tpu as pltpu

N = 8192
BR = 256
SUB = 8
NSUB = BR // SUB
GRID = N // BR
BIG = 3.0e38


def _body(emb_ref, r_ref, val_ref, idx_ref, vacc_ref, iacc_ref):
    i = pl.program_id(0)

    @pl.when(i == 0)
    def _init():
        vacc_ref[...] = jnp.full((SUB, N), -1.0, jnp.float32)
        iacc_ref[...] = jnp.zeros((SUB, N), jnp.float32)

    gid0 = (i * NSUB).astype(jnp.float32)
    for k in range(NSUB):
        x = r_ref[pl.ds(k * SUB, SUB), :] * emb_ref[pl.ds(k * SUB, SUB), :]
        m = x > vacc_ref[...]
        vacc_ref[...] = jnp.where(m, x, vacc_ref[...])
        iacc_ref[...] = jnp.where(m, gid0 + float(k), iacc_ref[...])

    @pl.when(i == GRID - 1)
    def _final():
        vacc = vacc_ref[...]
        sub = jax.lax.broadcasted_iota(jnp.int32, (SUB, N), 0)
        rowf = iacc_ref[...] * float(SUB) + sub.astype(jnp.float32)
        bmax = jnp.max(vacc, axis=0, keepdims=True)
        cand = jnp.where(vacc == bmax, rowf, BIG)
        bidx = jnp.min(cand, axis=0, keepdims=True)
        val_ref[...] = bmax
        idx_ref[...] = jnp.where(bmax > 0.0, bidx, 0.0)


def kernel(embedding, r_embedding):
    emb_t = embedding.reshape(N, 1)
    val, idx = pl.pallas_call(
        _body,
        grid=(GRID,),
        in_specs=[
            pl.BlockSpec((BR, 1), lambda i: (i, 0)),
            pl.BlockSpec((BR, N), lambda i: (i, 0)),
        ],
        out_specs=[
            pl.BlockSpec((1, N), lambda i: (0, 0)),
            pl.BlockSpec((1, N), lambda i: (0, 0)),
        ],
        out_shape=[
            jax.ShapeDtypeStruct((1, N), jnp.float32),
            jax.ShapeDtypeStruct((1, N), jnp.float32),
        ],
        scratch_shapes=[
            pltpu.VMEM((SUB, N), jnp.float32),
            pltpu.VMEM((SUB, N), jnp.float32),
        ],
    )(emb_t, r_embedding)
    return val, idx.reshape(N)
